# baseline (speedup 1.0000x reference)
import functools

import jax
import jax.numpy as jnp
from jax import lax
from jax.experimental import pallas as pl
from jax.experimental.pallas import tpu as pltpu
from jax.experimental.pallas import tpu_sc as plsc

_B, _N, _C, _H, _W = 2, 4, 96, 224, 224
_L = 16
_NUM_WORKERS = 32
_ROWS = _B * _H
_ROWS_PER_WORKER = _ROWS // _NUM_WORKERS
_GROUPS = _ROWS_PER_WORKER * (_W // _L)


def _sc_rank_body(ts_hbm, rank_hbm, ts_v, rank_v):
    wid = lax.axis_index("s") * 2 + lax.axis_index("c")
    consts = [jnp.full((_L,), m, jnp.int32) for m in range(_N)]
    zero, one = consts[0], consts[1]

    row0 = wid * _ROWS_PER_WORKER
    b = row0 // _H
    h0 = row0 % _H

    pltpu.sync_copy(ts_hbm.at[b, :, pl.ds(h0, _ROWS_PER_WORKER), :], ts_v)

    def group_body(fg, _):
        hh = fg // (_W // _L)
        sl = pl.ds((fg % (_W // _L)) * _L, _L)
        t = [ts_v[m, hh, sl] for m in range(_N)]
        for m in range(_N):
            r = zero
            for j in range(_N):
                if j == m:
                    continue
                r = r + jnp.where(t[j] > t[m], one, zero)
                if j < m:
                    r = r + jnp.where(t[j] == t[m], one, zero)
            rank_v[m, hh, sl] = r
        return 0

    lax.fori_loop(0, _GROUPS, group_body, 0)
    pltpu.sync_copy(rank_v,
                    rank_hbm.at[b, :, pl.ds(h0, _ROWS_PER_WORKER), :])


def _sc_ranks(timestamps):
    mesh = plsc.VectorSubcoreMesh(core_axis_name="c", subcore_axis_name="s")
    run = pl.kernel(
        _sc_rank_body,
        out_type=jax.ShapeDtypeStruct((_B, _N, _H, _W), jnp.int32),
        mesh=mesh,
        compiler_params=pltpu.CompilerParams(use_tc_tiling_on_sc=False),
        scratch_types=[
            pltpu.VMEM((_N, _ROWS_PER_WORKER, _W), jnp.float32),
            pltpu.VMEM((_N, _ROWS_PER_WORKER, _W), jnp.int32),
        ],
    )
    return run(timestamps)


_TC_CB = 8


def _tc_body(rank_ref, feat_ref, out_ref):
    ranks = [rank_ref[m] for m in range(_N)]
    masks = [[ranks[m] == n for m in range(_N - 1)] for n in range(_N)]
    for c in range(_TC_CB):
        f = [feat_ref[m, c] for m in range(_N)]
        for n in range(_N):
            v = f[_N - 1]
            for m in range(_N - 2, -1, -1):
                v = jnp.where(masks[n][m], f[m], v)
            out_ref[n, c] = v


def _tc_permute(features, ranks):
    b, n, c, h, w = features.shape
    grid = (b, c // _TC_CB)
    return pl.pallas_call(
        _tc_body,
        grid=grid,
        in_specs=[
            pl.BlockSpec((None, n, h, w), lambda bi, ci: (bi, 0, 0, 0)),
            pl.BlockSpec((None, n, _TC_CB, h, w),
                         lambda bi, ci: (bi, 0, ci, 0, 0)),
        ],
        out_specs=pl.BlockSpec((None, n, _TC_CB, h, w),
                               lambda bi, ci: (bi, 0, ci, 0, 0)),
        out_shape=jax.ShapeDtypeStruct(features.shape, features.dtype),
    )(ranks, features)


def kernel(features, timestamps, dim):
    del dim
    return _tc_permute(features, _sc_ranks(timestamps))


# speedup vs baseline: 1.0132x; 1.0132x over previous
import functools

import jax
import jax.numpy as jnp
from jax import lax
from jax.experimental import pallas as pl
from jax.experimental.pallas import tpu as pltpu
from jax.experimental.pallas import tpu_sc as plsc

_B, _N, _C, _H, _W = 2, 4, 96, 224, 224
_L = 16
_NUM_WORKERS = 32
_ROWS = _B * _H
_ROWS_PER_WORKER = _ROWS // _NUM_WORKERS
_GROUPS = _ROWS_PER_WORKER * (_W // _L)


def _sc_rank_body(ts_hbm, rank_hbm, ts_v, rank_v):
    wid = lax.axis_index("s") * 2 + lax.axis_index("c")
    consts = [jnp.full((_L,), m, jnp.int32) for m in range(_N)]
    zero, one = consts[0], consts[1]
    byte_scale = [jnp.full((_L,), 1 << (8 * m), jnp.int32) for m in range(_N)]

    row0 = wid * _ROWS_PER_WORKER
    b = row0 // _H
    h0 = row0 % _H

    pltpu.sync_copy(ts_hbm.at[b, :, pl.ds(h0, _ROWS_PER_WORKER), :], ts_v)

    def group_body(fg, _):
        hh = fg // (_W // _L)
        sl = pl.ds((fg % (_W // _L)) * _L, _L)
        t = [ts_v[m, hh, sl] for m in range(_N)]
        word = zero
        for m in range(_N):
            r = zero
            for j in range(_N):
                if j == m:
                    continue
                r = r + jnp.where(t[j] > t[m], one, zero)
                if j < m:
                    r = r + jnp.where(t[j] == t[m], one, zero)
            word = word + r * byte_scale[m]
        rank_v[hh, sl] = word
        return 0

    lax.fori_loop(0, _GROUPS, group_body, 0)
    pltpu.sync_copy(rank_v,
                    rank_hbm.at[b, pl.ds(h0, _ROWS_PER_WORKER), :])


def _sc_ranks(timestamps):
    mesh = plsc.VectorSubcoreMesh(core_axis_name="c", subcore_axis_name="s")
    run = pl.kernel(
        _sc_rank_body,
        out_type=jax.ShapeDtypeStruct((_B, _H, _W), jnp.int32),
        mesh=mesh,
        compiler_params=pltpu.CompilerParams(use_tc_tiling_on_sc=False),
        scratch_types=[
            pltpu.VMEM((_N, _ROWS_PER_WORKER, _W), jnp.float32),
            pltpu.VMEM((_ROWS_PER_WORKER, _W), jnp.int32),
        ],
    )
    return run(timestamps)


_TC_CB = 8


def _tc_body(rank_ref, feat_ref, out_ref):
    word = rank_ref[...]
    ranks = [(word >> (8 * m)) & 0xFF for m in range(_N - 1)]
    masks = [[ranks[m] == n for m in range(_N - 1)] for n in range(_N)]
    for c in range(_TC_CB):
        f = [feat_ref[m, c] for m in range(_N)]
        for n in range(_N):
            v = f[_N - 1]
            for m in range(_N - 2, -1, -1):
                v = jnp.where(masks[n][m], f[m], v)
            out_ref[n, c] = v


def _tc_permute(features, ranks):
    b, n, c, h, w = features.shape
    grid = (b, c // _TC_CB)
    return pl.pallas_call(
        _tc_body,
        grid=grid,
        in_specs=[
            pl.BlockSpec((None, h, w), lambda bi, ci: (bi, 0, 0)),
            pl.BlockSpec((None, n, _TC_CB, h, w),
                         lambda bi, ci: (bi, 0, ci, 0, 0)),
        ],
        out_specs=pl.BlockSpec((None, n, _TC_CB, h, w),
                               lambda bi, ci: (bi, 0, ci, 0, 0)),
        out_shape=jax.ShapeDtypeStruct(features.shape, features.dtype),
    )(ranks, features)


def kernel(features, timestamps, dim):
    del dim
    return _tc_permute(features, _sc_ranks(timestamps))


# speedup vs baseline: 1.0143x; 1.0011x over previous
import functools

import jax
import jax.numpy as jnp
from jax import lax
from jax.experimental import pallas as pl
from jax.experimental.pallas import tpu as pltpu
from jax.experimental.pallas import tpu_sc as plsc

_B, _N, _C, _H, _W = 2, 4, 96, 224, 224
_L = 16
_NUM_WORKERS = 32
_ROWS = _B * _H
_ROWS_PER_WORKER = _ROWS // _NUM_WORKERS
_GROUPS = _ROWS_PER_WORKER * (_W // _L)


def _sc_rank_body(ts_hbm, rank_hbm, ts_v, rank_v):
    wid = lax.axis_index("s") * 2 + lax.axis_index("c")
    consts = [jnp.full((_L,), m, jnp.int32) for m in range(_N)]
    zero, one = consts[0], consts[1]
    byte_scale = [jnp.full((_L,), 1 << (8 * m), jnp.int32) for m in range(_N)]

    row0 = wid * _ROWS_PER_WORKER
    b = row0 // _H
    h0 = row0 % _H

    pltpu.sync_copy(ts_hbm.at[b, :, pl.ds(h0, _ROWS_PER_WORKER), :], ts_v)

    def group_body(fg, _):
        hh = fg // (_W // _L)
        sl = pl.ds((fg % (_W // _L)) * _L, _L)
        t = [ts_v[m, hh, sl] for m in range(_N)]
        word = zero
        for m in range(_N):
            r = zero
            for j in range(_N):
                if j == m:
                    continue
                r = r + jnp.where(t[j] > t[m], one, zero)
                if j < m:
                    r = r + jnp.where(t[j] == t[m], one, zero)
            word = word + r * byte_scale[m]
        rank_v[hh, sl] = word
        return 0

    lax.fori_loop(0, _GROUPS, group_body, 0)
    pltpu.sync_copy(rank_v,
                    rank_hbm.at[b, pl.ds(h0, _ROWS_PER_WORKER), :])


def _sc_ranks(timestamps):
    mesh = plsc.VectorSubcoreMesh(core_axis_name="c", subcore_axis_name="s")
    run = pl.kernel(
        _sc_rank_body,
        out_type=jax.ShapeDtypeStruct((_B, _H, _W), jnp.int32),
        mesh=mesh,
        compiler_params=pltpu.CompilerParams(use_tc_tiling_on_sc=False),
        scratch_types=[
            pltpu.VMEM((_N, _ROWS_PER_WORKER, _W), jnp.float32),
            pltpu.VMEM((_ROWS_PER_WORKER, _W), jnp.int32),
        ],
    )
    return run(timestamps)


_TC_CB = 16


def _tc_body(rank_ref, feat_ref, out_ref):
    word = rank_ref[...]
    ranks = [(word >> (8 * m)) & 0xFF for m in range(_N - 1)]
    masks = [[ranks[m] == n for m in range(_N - 1)] for n in range(_N)]
    for c in range(_TC_CB):
        f = [feat_ref[m, c] for m in range(_N)]
        for n in range(_N):
            v = f[_N - 1]
            for m in range(_N - 2, -1, -1):
                v = jnp.where(masks[n][m], f[m], v)
            out_ref[n, c] = v


def _tc_permute(features, ranks):
    b, n, c, h, w = features.shape
    grid = (b, c // _TC_CB)
    return pl.pallas_call(
        _tc_body,
        grid=grid,
        in_specs=[
            pl.BlockSpec((None, h, w), lambda bi, ci: (bi, 0, 0)),
            pl.BlockSpec((None, n, _TC_CB, h, w),
                         lambda bi, ci: (bi, 0, ci, 0, 0)),
        ],
        out_specs=pl.BlockSpec((None, n, _TC_CB, h, w),
                               lambda bi, ci: (bi, 0, ci, 0, 0)),
        out_shape=jax.ShapeDtypeStruct(features.shape, features.dtype),
    )(ranks, features)


def kernel(features, timestamps, dim):
    del dim
    return _tc_permute(features, _sc_ranks(timestamps))
